# baseline (speedup 1.0000x reference)
import functools

import jax
import jax.numpy as jnp
from jax import lax
from jax.experimental import pallas as pl
from jax.experimental.pallas import tpu as pltpu
from jax.experimental.pallas import tpu_sc as plsc

VOCAB = 1000
EMB = 60
EMBP = 128
B = 16384
L = 20
LP = 24
TOTAL = B * LP

_INFO = plsc.get_sparse_core_info()
NC = _INFO.num_cores
NS = _INFO.num_subcores
NW = NC * NS
PER_W = TOTAL // NW
SUB = 128
NSUB = 2
BLK = SUB * NSUB
IDX_ROWS = 8
SUPER = SUB * IDX_ROWS
BPS = SUPER // BLK
NSUPER = PER_W // SUPER


def _matmul_body(a_ref, x_ref, o_ref):
    o_ref[...] = jnp.dot(a_ref[...], x_ref[...],
                         preferred_element_type=jnp.float32)


def _propagate(A, xp):
    return pl.pallas_call(
        _matmul_body,
        out_shape=jax.ShapeDtypeStruct((VOCAB, EMBP), jnp.float32),
    )(A, xp)


@functools.partial(
    pl.kernel,
    mesh=plsc.VectorSubcoreMesh(core_axis_name="c", subcore_axis_name="s"),
    out_type=jax.ShapeDtypeStruct((TOTAL, EMBP), jnp.float32),
    scratch_types=[
        pltpu.VMEM((2, IDX_ROWS, SUB), jnp.int32),
        pltpu.VMEM((2, BLK, EMBP), jnp.float32),
        [pltpu.SemaphoreType.DMA] * 2,
        [pltpu.SemaphoreType.DMA] * 2,
    ],
)
def _gather(xg_hbm, idx_hbm, out_hbm, idx_v, rows_v, sem_g, sem_w):
    wid = lax.axis_index("s") * NC + lax.axis_index("c")
    base_w = wid * PER_W
    row0_w = base_w // SUB

    def fire_gathers(r, slot, part):
        for j in range(NSUB):
            pltpu.async_copy(
                xg_hbm.at[idx_v.at[slot, part * NSUB + j]],
                rows_v.at[r, pl.ds(j * SUB, SUB)],
                sem_g[r],
            )

    def drain_gathers(r):
        for j in range(NSUB):
            pltpu.make_async_copy(
                xg_hbm.at[idx_v.at[0, 0]],
                rows_v.at[r, pl.ds(j * SUB, SUB)],
                sem_g[r],
            ).wait()

    def fire_write(r, s, k):
        pltpu.async_copy(
            rows_v.at[r],
            out_hbm.at[pl.ds(base_w + s * SUPER + k * BLK, BLK)],
            sem_w[r],
        )

    def drain_write(r):
        pltpu.make_async_copy(
            rows_v.at[r],
            out_hbm.at[pl.ds(base_w, BLK)],
            sem_w[r],
        ).wait()

    def fetch_idx(s, slot):
        pltpu.sync_copy(
            idx_hbm.at[pl.ds(pl.multiple_of(row0_w + s * IDX_ROWS, IDX_ROWS),
                             IDX_ROWS)],
            idx_v.at[slot],
        )

    fetch_idx(0, 0)
    fire_gathers(0, 0, 0)

    def body(s, carry):
        q = lax.rem(s, 2)
        qn = lax.rem(s + 1, 2)

        @pl.when(s < NSUPER - 1)
        def _():
            fetch_idx(s + 1, qn)

        for k in range(BPS):
            r = k % 2
            drain_gathers(r)
            if k == 0:
                @pl.when(s > 0)
                def _():
                    drain_write(1)
            else:
                drain_write(1 - r)
            if k < BPS - 1:
                fire_gathers(1 - r, q, k + 1)
            else:
                @pl.when(s < NSUPER - 1)
                def _():
                    fire_gathers(1 - r, qn, 0)
            fire_write(r, s, k)
        return carry

    lax.fori_loop(0, NSUPER, body, 0)
    drain_write(1)


def kernel(A, x, movie_ids):
    xp = jnp.pad(x, ((0, 0), (0, EMBP - EMB)))
    xg = _propagate(A, xp)
    idx = jnp.pad(movie_ids.astype(jnp.int32), ((0, 0), (0, LP - L)))
    idx = idx.reshape(TOTAL // SUB, SUB)
    out = _gather(xg, idx)
    return out.reshape(B, LP, EMBP)[:, :L, :EMB]


# speedup vs baseline: 8.9282x; 8.9282x over previous
import functools

import jax
import jax.numpy as jnp
from jax import lax
from jax.experimental import pallas as pl
from jax.experimental.pallas import tpu as pltpu
from jax.experimental.pallas import tpu_sc as plsc

VOCAB = 1000
EMB = 60
EMBP = 128
B = 16384
L = 20
LP = 24
TOTAL = B * LP

_INFO = plsc.get_sparse_core_info()
NC = _INFO.num_cores
NS = _INFO.num_subcores
NW = NC * NS
PER_W = TOTAL // NW
SUB = 128
NSUB = 2
BLK = SUB * NSUB
IDX_ROWS = 8
SUPER = SUB * IDX_ROWS
BPS = SUPER // BLK
NSUPER = PER_W // SUPER


def _matmul_body(a_ref, x_ref, o_ref):
    o_ref[...] = jnp.dot(a_ref[...], x_ref[...],
                         preferred_element_type=jnp.float32)


def _propagate(A, xp):
    return pl.pallas_call(
        _matmul_body,
        out_shape=jax.ShapeDtypeStruct((VOCAB, EMBP), jnp.float32),
    )(A, xp)


@functools.partial(
    pl.kernel,
    mesh=plsc.VectorSubcoreMesh(core_axis_name="c", subcore_axis_name="s"),
    out_type=jax.ShapeDtypeStruct((TOTAL, EMBP), jnp.float32),
    scratch_types=[
        pltpu.VMEM((2, IDX_ROWS, SUB), jnp.int32),
        pltpu.VMEM((2, BLK, EMBP), jnp.float32),
        [pltpu.SemaphoreType.DMA] * 2,
        [pltpu.SemaphoreType.DMA] * 2,
    ],
)
def _gather(xg_hbm, idx_hbm, out_hbm, idx_v, rows_v, sem_g, sem_w):
    wid = lax.axis_index("s") * NC + lax.axis_index("c")
    base_w = wid * PER_W
    row0_w = base_w // SUB

    def fire_gathers(r, slot, part):
        for j in range(NSUB):
            pltpu.async_copy(
                xg_hbm.at[idx_v.at[slot, part * NSUB + j]],
                rows_v.at[r, pl.ds(j * SUB, SUB)],
                sem_g[r],
            )

    def drain_gathers(r):
        for j in range(NSUB):
            pltpu.make_async_copy(
                xg_hbm.at[idx_v.at[0, 0]],
                rows_v.at[r, pl.ds(j * SUB, SUB)],
                sem_g[r],
            ).wait()

    def fire_write(r, s, k):
        pltpu.async_copy(
            rows_v.at[r],
            out_hbm.at[pl.ds(base_w + s * SUPER + k * BLK, BLK)],
            sem_w[r],
        )

    def drain_write(r):
        pltpu.make_async_copy(
            rows_v.at[r],
            out_hbm.at[pl.ds(base_w, BLK)],
            sem_w[r],
        ).wait()

    def fetch_idx(s, slot):
        pltpu.sync_copy(
            idx_hbm.at[pl.ds(pl.multiple_of(row0_w + s * IDX_ROWS, IDX_ROWS),
                             IDX_ROWS)],
            idx_v.at[slot],
        )

    fetch_idx(0, 0)
    fire_gathers(0, 0, 0)

    def body(s, carry):
        q = lax.rem(s, 2)
        qn = lax.rem(s + 1, 2)

        @pl.when(s < NSUPER - 1)
        def _():
            fetch_idx(s + 1, qn)

        for k in range(BPS):
            r = k % 2
            drain_gathers(r)
            if k == 0:
                @pl.when(s > 0)
                def _():
                    drain_write(1)
            else:
                drain_write(1 - r)
            if k < BPS - 1:
                fire_gathers(1 - r, q, k + 1)
            else:
                @pl.when(s < NSUPER - 1)
                def _():
                    fire_gathers(1 - r, qn, 0)
            fire_write(r, s, k)
        return carry

    lax.fori_loop(0, NSUPER, body, 0)
    drain_write(1)


def kernel(A, x, movie_ids):
    xp = jnp.pad(x, ((0, 0), (0, EMBP - EMB)))
    xg = _propagate(A, xp)
    pad_vals = (jnp.arange(B * (LP - L), dtype=jnp.int32) % VOCAB)
    idx = jnp.concatenate(
        [movie_ids.astype(jnp.int32), pad_vals.reshape(B, LP - L)], axis=1)
    idx = idx.reshape(TOTAL // SUB, SUB)
    out = _gather(xg, idx)
    return out.reshape(B, LP, EMBP)[:, :L, :EMB]


# speedup vs baseline: 12.2649x; 1.3737x over previous
import functools

import jax
import jax.numpy as jnp
from jax import lax
from jax.experimental import pallas as pl
from jax.experimental.pallas import tpu as pltpu
from jax.experimental.pallas import tpu_sc as plsc

VOCAB = 1000
EMB = 60
EMBP = 64
B = 16384
L = 20
LANES = 16

_INFO = plsc.get_sparse_core_info()
NC = _INFO.num_cores
NS = _INFO.num_subcores
NW = NC * NS
NBT = B // 128
UNITS = L * NBT
UPW = UNITS // NW
NGRP = 128 // LANES
NSUPER = UPW // 8


def _matmul_body(a_ref, x_ref, o_ref):
    o_ref[...] = jnp.dot(a_ref[...], x_ref[...],
                         preferred_element_type=jnp.float32)


def _propagate(A, x):
    return pl.pallas_call(
        _matmul_body,
        out_shape=jax.ShapeDtypeStruct((VOCAB, EMB), jnp.float32),
    )(A, x)


@functools.partial(
    pl.kernel,
    mesh=plsc.VectorSubcoreMesh(core_axis_name="c", subcore_axis_name="s"),
    out_type=jax.ShapeDtypeStruct((L, EMBP, B), jnp.float32),
    scratch_types=[
        pltpu.VMEM((VOCAB * EMB,), jnp.float32),
        pltpu.VMEM((8, 128), jnp.int32),
        pltpu.VMEM((2, EMBP, 128), jnp.float32),
        [pltpu.SemaphoreType.DMA] * 2,
    ],
    compiler_params=pltpu.CompilerParams(needs_layout_passes=False),
)
def _gather(xg_hbm, ids_hbm, out_hbm, table_v, idx_v, stage_v, sem_w):
    wid = lax.axis_index("s") * NC + lax.axis_index("c")
    u0 = wid * UPW

    pltpu.sync_copy(xg_hbm, table_v)

    def drain_write(p):
        pltpu.make_async_copy(
            stage_v.at[p],
            out_hbm.at[0, pl.ds(0, EMBP), pl.ds(0, 128)],
            sem_w[p],
        ).wait()

    def do_unit(l, bt, j, p):
        def grp(g, carry):
            goff = pl.multiple_of(g * LANES, LANES)
            ptr = idx_v[j, pl.ds(goff, LANES)]
            for e in range(EMB):
                stage_v[p, e, pl.ds(goff, LANES)] = (
                    plsc.load_gather(table_v, [ptr]))
                ptr = ptr + 1
            return carry

        lax.fori_loop(0, NGRP, grp, 0)
        b_off = pl.multiple_of((bt + j) * 128, 128)
        pltpu.async_copy(
            stage_v.at[p],
            out_hbm.at[l, pl.ds(0, EMBP), pl.ds(b_off, 128)],
            sem_w[p],
        )

    def super_body(s, carry):
        n0 = u0 + 8 * s
        l = lax.div(n0, NBT)
        bt0 = lax.rem(n0, NBT)
        row0 = pl.multiple_of(bt0, 8)
        pltpu.sync_copy(ids_hbm.at[l, pl.ds(row0, 8)], idx_v)

        def pair(i2, carry2):
            @pl.when(s + i2 > 0)
            def _():
                drain_write(0)
            do_unit(l, bt0, 2 * i2, 0)

            @pl.when(s + i2 > 0)
            def _():
                drain_write(1)
            do_unit(l, bt0, 2 * i2 + 1, 1)
            return carry2

        lax.fori_loop(0, 4, pair, 0)
        return carry

    lax.fori_loop(0, NSUPER, super_body, 0)
    drain_write(0)
    drain_write(1)


def kernel(A, x, movie_ids):
    xg = _propagate(A, x)
    xg_flat = xg.reshape(VOCAB * EMB)
    ids_sc = (movie_ids.astype(jnp.int32) * EMB).T
    p = _gather(xg_flat, ids_sc.reshape(L, NBT, 128))
    return p.transpose(2, 0, 1)[:, :, :EMB]


# speedup vs baseline: 14.6923x; 1.1979x over previous
import functools

import jax
import jax.numpy as jnp
from jax import lax
from jax.experimental import pallas as pl
from jax.experimental.pallas import tpu as pltpu
from jax.experimental.pallas import tpu_sc as plsc

VOCAB = 1000
EMB = 60
STRIDE = 61
EMBP = 64
B = 16384
L = 20
LANES = 16

_INFO = plsc.get_sparse_core_info()
NC = _INFO.num_cores
NS = _INFO.num_subcores
NW = NC * NS
NBT = B // 128
UNITS = L * NBT
UPW = UNITS // NW
NGRP = 128 // LANES
NSUPER = UPW // 8


def _matmul_body(a_ref, x_ref, o_ref):
    o_ref[...] = jnp.dot(a_ref[...], x_ref[...],
                         preferred_element_type=jnp.float32)


def _propagate(A, x):
    return pl.pallas_call(
        _matmul_body,
        out_shape=jax.ShapeDtypeStruct((VOCAB, EMB), jnp.float32),
    )(A, x)


@functools.partial(
    pl.kernel,
    mesh=plsc.VectorSubcoreMesh(core_axis_name="c", subcore_axis_name="s"),
    out_type=jax.ShapeDtypeStruct((L, EMBP, B), jnp.float32),
    scratch_types=[
        pltpu.VMEM((VOCAB * STRIDE,), jnp.float32),
        pltpu.VMEM((8, 128), jnp.int32),
        pltpu.VMEM((2, EMBP, 128), jnp.float32),
        [pltpu.SemaphoreType.DMA] * 2,
    ],
    compiler_params=pltpu.CompilerParams(needs_layout_passes=False),
)
def _gather(xg_hbm, ids_hbm, out_hbm, table_v, idx_v, stage_v, sem_w):
    wid = lax.axis_index("s") * NC + lax.axis_index("c")
    u0 = wid * UPW

    pltpu.sync_copy(xg_hbm, table_v)

    def drain_write(p):
        pltpu.make_async_copy(
            stage_v.at[p],
            out_hbm.at[0, pl.ds(0, EMBP), pl.ds(0, 128)],
            sem_w[p],
        ).wait()

    def do_unit(l, bt, j, p):
        def grp(g, carry):
            goff = pl.multiple_of(g * LANES, LANES)
            ptr0 = idx_v[j, pl.ds(goff, LANES)]
            for e in range(EMB):
                stage_v[p, e, pl.ds(goff, LANES)] = (
                    plsc.load_gather(table_v, [ptr0 + e]))
            return carry

        lax.fori_loop(0, NGRP, grp, 0)
        b_off = pl.multiple_of((bt + j) * 128, 128)
        pltpu.async_copy(
            stage_v.at[p],
            out_hbm.at[l, pl.ds(0, EMBP), pl.ds(b_off, 128)],
            sem_w[p],
        )

    def super_body(s, carry):
        n0 = u0 + 8 * s
        l = lax.div(n0, NBT)
        bt0 = lax.rem(n0, NBT)
        row0 = pl.multiple_of(bt0, 8)
        pltpu.sync_copy(ids_hbm.at[l, pl.ds(row0, 8)], idx_v)

        def pair(i2, carry2):
            @pl.when(s + i2 > 0)
            def _():
                drain_write(0)
            do_unit(l, bt0, 2 * i2, 0)

            @pl.when(s + i2 > 0)
            def _():
                drain_write(1)
            do_unit(l, bt0, 2 * i2 + 1, 1)
            return carry2

        lax.fori_loop(0, 4, pair, 0)
        return carry

    lax.fori_loop(0, NSUPER, super_body, 0)
    drain_write(0)
    drain_write(1)


def kernel(A, x, movie_ids):
    xg = _propagate(A, x)
    xg_flat = jnp.pad(xg, ((0, 0), (0, STRIDE - EMB))).reshape(VOCAB * STRIDE)
    ids_sc = (movie_ids.astype(jnp.int32) * STRIDE).T
    p = _gather(xg_flat, ids_sc.reshape(L, NBT, 128))
    return p.transpose(2, 0, 1)[:, :, :EMB]


# speedup vs baseline: 34.3713x; 2.3394x over previous
import functools

import jax
import jax.numpy as jnp
from jax import lax
from jax.experimental import pallas as pl
from jax.experimental.pallas import tpu as pltpu
from jax.experimental.pallas import tpu_sc as plsc

VOCAB = 1000
EMB = 60
STRIDE = 61
EMBP = 64
B = 16384
L = 20
LANES = 16

_INFO = plsc.get_sparse_core_info()
NC = _INFO.num_cores
NS = _INFO.num_subcores
NW = NC * NS
NBT = B // 128
UNITS = L * NBT
UPW = UNITS // NW
NGRP = 128 // LANES
NSUPER = UPW // 8


def _matmul_body(a_ref, x_ref, o_ref):
    o_ref[...] = jnp.dot(a_ref[...], x_ref[...],
                         preferred_element_type=jnp.float32)


def _propagate(A, x):
    return pl.pallas_call(
        _matmul_body,
        out_shape=jax.ShapeDtypeStruct((VOCAB, EMB), jnp.float32),
    )(A, x)


@functools.partial(
    pl.kernel,
    mesh=plsc.VectorSubcoreMesh(core_axis_name="c", subcore_axis_name="s"),
    out_type=jax.ShapeDtypeStruct((L, EMBP, B), jnp.float32),
    scratch_types=[
        pltpu.VMEM((VOCAB * STRIDE,), jnp.float32),
        pltpu.VMEM((8, 128), jnp.int32),
        pltpu.VMEM((2, EMBP, 128), jnp.float32),
        [pltpu.SemaphoreType.DMA] * 2,
    ],
    compiler_params=pltpu.CompilerParams(needs_layout_passes=False),
)
def _gather(xg_hbm, ids_hbm, out_hbm, table_v, idx_v, stage_v, sem_w):
    wid = lax.axis_index("s") * NC + lax.axis_index("c")
    u0 = wid * UPW

    pltpu.sync_copy(xg_hbm, table_v)

    def drain_write(p):
        pltpu.make_async_copy(
            stage_v.at[p],
            out_hbm.at[0, pl.ds(0, EMBP), pl.ds(0, 128)],
            sem_w[p],
        ).wait()

    def do_unit(l, bt, j, p):
        def grp(g, carry):
            goff = pl.multiple_of(g * LANES, LANES)
            ptr0 = idx_v[j, pl.ds(goff, LANES)]
            for e0 in range(0, EMB, 10):
                vals = [plsc.load_gather(table_v, [ptr0 + (e0 + t)])
                        for t in range(10)]
                for t in range(10):
                    stage_v[p, e0 + t, pl.ds(goff, LANES)] = vals[t]
            return carry

        lax.fori_loop(0, NGRP, grp, 0)
        b_off = pl.multiple_of((bt + j) * 128, 128)
        pltpu.async_copy(
            stage_v.at[p],
            out_hbm.at[l, pl.ds(0, EMBP), pl.ds(b_off, 128)],
            sem_w[p],
        )

    def super_body(s, carry):
        n0 = u0 + 8 * s
        l = lax.div(n0, NBT)
        bt0 = lax.rem(n0, NBT)
        row0 = pl.multiple_of(bt0, 8)
        pltpu.sync_copy(ids_hbm.at[l, pl.ds(row0, 8)], idx_v)

        def pair(i2, carry2):
            @pl.when(s + i2 > 0)
            def _():
                drain_write(0)
            do_unit(l, bt0, 2 * i2, 0)

            @pl.when(s + i2 > 0)
            def _():
                drain_write(1)
            do_unit(l, bt0, 2 * i2 + 1, 1)
            return carry2

        lax.fori_loop(0, 4, pair, 0)
        return carry

    lax.fori_loop(0, NSUPER, super_body, 0)
    drain_write(0)
    drain_write(1)


def kernel(A, x, movie_ids):
    xg = _propagate(A, x)
    xg_flat = jnp.pad(xg, ((0, 0), (0, STRIDE - EMB))).reshape(VOCAB * STRIDE)
    ids_sc = (movie_ids.astype(jnp.int32) * STRIDE).T
    p = _gather(xg_flat, ids_sc.reshape(L, NBT, 128))
    return p.transpose(2, 0, 1)[:, :, :EMB]


# speedup vs baseline: 37.3701x; 1.0872x over previous
import functools

import jax
import jax.numpy as jnp
from jax import lax
from jax.experimental import pallas as pl
from jax.experimental.pallas import tpu as pltpu
from jax.experimental.pallas import tpu_sc as plsc

VOCAB = 1000
EMB = 60
STRIDE = 61
EMBP = 64
B = 16384
L = 20
LANES = 16

_INFO = plsc.get_sparse_core_info()
NC = _INFO.num_cores
NS = _INFO.num_subcores
NW = NC * NS
NBT = B // 128
UNITS = L * NBT
UPW = UNITS // NW
NGRP = 128 // LANES
NSUPER = UPW // 8


def _matmul_body(a_ref, x_ref, o_ref):
    o_ref[...] = jnp.dot(a_ref[...], x_ref[...],
                         preferred_element_type=jnp.float32)


def _propagate(A, x):
    return pl.pallas_call(
        _matmul_body,
        out_shape=jax.ShapeDtypeStruct((VOCAB, EMB), jnp.float32),
    )(A, x)


@functools.partial(
    pl.kernel,
    mesh=plsc.VectorSubcoreMesh(core_axis_name="c", subcore_axis_name="s"),
    out_type=jax.ShapeDtypeStruct((L, EMBP, B), jnp.float32),
    scratch_types=[
        pltpu.VMEM((VOCAB * STRIDE,), jnp.float32),
        pltpu.VMEM((2, 8, 128), jnp.int32),
        pltpu.VMEM((4, EMBP, 128), jnp.float32),
        [pltpu.SemaphoreType.DMA] * 4,
        [pltpu.SemaphoreType.DMA] * 2,
    ],
    compiler_params=pltpu.CompilerParams(needs_layout_passes=False),
)
def _gather(xg_hbm, ids_hbm, out_hbm, table_v, idx_v, stage_v, sem_w, sem_i):
    wid = lax.axis_index("s") * NC + lax.axis_index("c")
    u0 = wid * UPW

    def fire_idx(s, slot):
        n0 = u0 + 8 * s
        l = lax.div(n0, NBT)
        row0 = pl.multiple_of(lax.rem(n0, NBT), 8)
        pltpu.async_copy(
            ids_hbm.at[l, pl.ds(row0, 8)], idx_v.at[slot], sem_i[slot])

    def wait_idx(slot):
        pltpu.make_async_copy(
            ids_hbm.at[0, pl.ds(0, 8)], idx_v.at[slot], sem_i[slot]).wait()

    def drain_write(p):
        pltpu.make_async_copy(
            stage_v.at[p],
            out_hbm.at[0, pl.ds(0, EMBP), pl.ds(0, 128)],
            sem_w[p],
        ).wait()

    fire_idx(0, 0)
    pltpu.sync_copy(xg_hbm, table_v)

    def do_unit(l, bt, slot, j, p):
        def grp(g, carry):
            goff = pl.multiple_of(g * LANES, LANES)
            ptr0 = idx_v[slot, j, pl.ds(goff, LANES)]
            for e0 in range(0, EMB, 10):
                vals = [plsc.load_gather(table_v, [ptr0 + (e0 + t)])
                        for t in range(10)]
                for t in range(10):
                    stage_v[p, e0 + t, pl.ds(goff, LANES)] = vals[t]
            return carry

        lax.fori_loop(0, NGRP, grp, 0)
        b_off = pl.multiple_of((bt + j) * 128, 128)
        pltpu.async_copy(
            stage_v.at[p],
            out_hbm.at[l, pl.ds(0, EMBP), pl.ds(b_off, 128)],
            sem_w[p],
        )

    def super_body(s, carry):
        q = lax.rem(s, 2)
        n0 = u0 + 8 * s
        l = lax.div(n0, NBT)
        bt0 = lax.rem(n0, NBT)

        @pl.when((s < NSUPER - 1) & (q == 0))
        def _():
            fire_idx(s + 1, 1)

        @pl.when((s < NSUPER - 1) & (q == 1))
        def _():
            fire_idx(s + 1, 0)

        @pl.when(q == 0)
        def _():
            wait_idx(0)

        @pl.when(q == 1)
        def _():
            wait_idx(1)

        for j in range(8):
            p = j % 4
            if j < 4:
                @pl.when(s > 0)
                def _():
                    drain_write(p)
            else:
                drain_write(p)
            do_unit(l, bt0, q, j, p)
        return carry

    lax.fori_loop(0, NSUPER, super_body, 0)
    for p in range(4):
        drain_write(p)


def kernel(A, x, movie_ids):
    xg = _propagate(A, x)
    xg_flat = jnp.pad(xg, ((0, 0), (0, STRIDE - EMB))).reshape(VOCAB * STRIDE)
    ids_sc = (movie_ids.astype(jnp.int32) * STRIDE).T
    p = _gather(xg_flat, ids_sc.reshape(L, NBT, 128))
    return p.transpose(2, 0, 1)[:, :, :EMB]
